# baseline (speedup 1.0000x reference)
import functools

import jax
import jax.numpy as jnp
from jax import lax
from jax.experimental import pallas as pl
from jax.experimental.pallas import tpu as pltpu
from jax.experimental.pallas import tpu_sc as plsc

_NC = 2
_NS = 16
_NW = _NC * _NS
_L = 16


@functools.cache
def _sc_pool(B, S, V, E):
    bpw = B // _NW
    nvec = E // _L
    mesh = plsc.VectorSubcoreMesh(core_axis_name="c", subcore_axis_name="s")

    @functools.partial(
        pl.kernel,
        out_type=jax.ShapeDtypeStruct((B, E), jnp.float32),
        mesh=mesh,
        scratch_types=[
            pltpu.VMEM((S, bpw), jnp.int32),
            pltpu.VMEM((bpw, E), jnp.float32),
            pltpu.VMEM((bpw, E), jnp.float32),
            pltpu.VMEM((bpw, E), jnp.float32),
            pltpu.SemaphoreType.DMA,
            pltpu.SemaphoreType.DMA,
        ],
        compiler_params=pltpu.CompilerParams(use_tc_tiling_on_sc=False),
    )
    def pool(table_hbm, text_hbm, out_hbm, idx_v, rows_a, rows_b, acc_v, sem_a, sem_b):
        wid = lax.axis_index("s") * _NC + lax.axis_index("c")
        base = wid * bpw
        pltpu.sync_copy(text_hbm.at[:, pl.ds(base, bpw)], idx_v)

        pltpu.async_copy(table_hbm.at[idx_v.at[0]], rows_a, sem_a)
        pltpu.async_copy(table_hbm.at[idx_v.at[1]], rows_b, sem_b)

        def zero(e, _):
            for j in range(nvec):
                acc_v[e, pl.ds(j * _L, _L)] = jnp.zeros((_L,), jnp.float32)
            return 0

        lax.fori_loop(0, bpw, zero, 0, unroll=8)

        def process(s, rows, sem):
            pltpu.make_async_copy(table_hbm.at[idx_v.at[s]], rows, sem).wait()

            def add(e, _):
                for j in range(nvec):
                    plsc.addupdate(
                        acc_v.at[e, pl.ds(j * _L, _L)],
                        rows[e, pl.ds(j * _L, _L)],
                    )
                return 0

            lax.fori_loop(0, bpw, add, 0, unroll=8)

            nxt = s + 2

            @pl.when(nxt < S)
            def _():
                pltpu.async_copy(table_hbm.at[idx_v.at[nxt]], rows, sem)

        def outer(k, _):
            process(2 * k, rows_a, sem_a)
            process(2 * k + 1, rows_b, sem_b)
            return 0

        lax.fori_loop(0, S // 2, outer, 0)
        pltpu.sync_copy(acc_v, out_hbm.at[pl.ds(base, bpw)])

    return pool


@functools.cache
def _tc_head(B, E, O, S):
    def head(p_ref, w_ref, b_ref, o_ref):
        pooled = p_ref[...] * (1.0 / S)
        o_ref[...] = (
            lax.dot_general(
                pooled, w_ref[...],
                (((1,), (1,)), ((), ())),
                preferred_element_type=jnp.float32,
            )
            + b_ref[...]
        )

    return pl.pallas_call(
        head,
        out_shape=jax.ShapeDtypeStruct((B, O), jnp.float32),
    )


def kernel(text, emb_table, W, b):
    S, B = text.shape
    V, E = emb_table.shape
    O = W.shape[0]
    pooled_sum = _sc_pool(B, S, V, E)(emb_table, text.astype(jnp.int32))
    return _tc_head(B, E, O, S)(pooled_sum, W, b.reshape(1, O))
